# Initial kernel scaffold; baseline (speedup 1.0000x reference)
import jax
import jax.numpy as jnp
from jax.experimental import pallas as pl


def kernel(rules, W):
    raise NotImplementedError("write your pallas kernel here")


# speedup vs baseline: 243.1840x; 243.1840x over previous
import jax
import jax.numpy as jnp
from jax import lax
from jax.experimental import pallas as pl
from jax.experimental.pallas import tpu as pltpu
from jax.experimental.pallas import tpu_sc as plsc

_B = 16384
_L = 200
_PAD = 1000
_TAB = 1008
_NC = 2
_NS = 16
_NW = _NC * _NS
_ROWS = _B // _NW
_LN = 16


def _noisy_or_body(rules_hbm, w_hbm, out_hbm, rules_v, tab_v, out_v, sem):
    wid = lax.axis_index("s") * _NC + lax.axis_index("c")
    base = wid * _ROWS

    cp = pltpu.make_async_copy(
        rules_hbm.at[pl.ds(base * _L, _ROWS * _L)], rules_v, sem
    )
    cp.start()

    pltpu.sync_copy(w_hbm, tab_v)

    def tbuild(j, c):
        w = tab_v[pl.ds(j * _LN, _LN)]
        p = 1.0 / (1.0 + jnp.exp(w))
        gidx = j * _LN + lax.broadcasted_iota(jnp.int32, (_LN,), 0)
        tab_v[pl.ds(j * _LN, _LN)] = jnp.where(gidx == _PAD, 1.0, p)
        return c

    lax.fori_loop(0, _TAB // _LN, tbuild, 0)

    cp.wait()

    def group(g, c):
        addr0 = (g * _LN + lax.broadcasted_iota(jnp.int32, (_LN,), 0)) * _L

        def body(i, carry):
            acc, addr = carry
            ids = plsc.load_gather(rules_v, [addr])
            vals = plsc.load_gather(tab_v, [ids])
            return acc * vals, addr + 1

        acc, _ = lax.fori_loop(
            0, _L, body, (jnp.full((_LN,), 1.0, jnp.float32), addr0)
        )
        out_v[pl.ds(g * _LN, _LN)] = jnp.clip(1.0 - acc, 1e-4, 1.0 - 1e-5)
        return c

    lax.fori_loop(0, _ROWS // _LN, group, 0)

    pltpu.sync_copy(out_v, out_hbm.at[pl.ds(base, _ROWS)])


def kernel(rules, W):
    rules_flat = rules.reshape(-1).astype(jnp.int32)
    wp = jnp.concatenate(
        [W.reshape(-1).astype(jnp.float32),
         jnp.zeros((_TAB - _PAD - 1,), jnp.float32)]
    )
    f = pl.kernel(
        _noisy_or_body,
        mesh=plsc.VectorSubcoreMesh(core_axis_name="c", subcore_axis_name="s"),
        compiler_params=pltpu.CompilerParams(needs_layout_passes=False),
        out_type=jax.ShapeDtypeStruct((_B,), jnp.float32),
        scratch_types=[
            pltpu.VMEM((_ROWS * _L,), jnp.int32),
            pltpu.VMEM((_TAB,), jnp.float32),
            pltpu.VMEM((_ROWS,), jnp.float32),
            pltpu.SemaphoreType.DMA,
        ],
    )
    return f(rules_flat, wp).reshape(_B, 1)


# speedup vs baseline: 272.2632x; 1.1196x over previous
import jax
import jax.numpy as jnp
from jax import lax
from jax.experimental import pallas as pl
from jax.experimental.pallas import tpu as pltpu
from jax.experimental.pallas import tpu_sc as plsc

_B = 16384
_L = 200
_PAD = 1000
_TAB = 1008
_NC = 2
_NS = 16
_NW = _NC * _NS
_ROWS = _B // _NW
_LN = 16


def _noisy_or_body(rules_hbm, w_hbm, out_hbm, rules_v, tab_v, out_v, sem):
    wid = lax.axis_index("s") * _NC + lax.axis_index("c")
    base = wid * _ROWS

    cp = pltpu.make_async_copy(
        rules_hbm.at[pl.ds(base * _L, _ROWS * _L)], rules_v, sem
    )
    cp.start()

    pltpu.sync_copy(w_hbm, tab_v)

    def tbuild(j, c):
        w = tab_v[pl.ds(j * _LN, _LN)]
        p = 1.0 / (1.0 + jnp.exp(w))
        gidx = j * _LN + lax.broadcasted_iota(jnp.int32, (_LN,), 0)
        tab_v[pl.ds(j * _LN, _LN)] = jnp.where(gidx == _PAD, 1.0, p)
        return c

    lax.fori_loop(0, _TAB // _LN, tbuild, 0)

    cp.wait()

    def group(g, c):
        addr0 = (g * _LN + lax.broadcasted_iota(jnp.int32, (_LN,), 0)) * _L
        accs = [None] * 8
        for i in range(_L):
            ids = plsc.load_gather(rules_v, [addr0 + i])
            vals = plsc.load_gather(tab_v, [ids])
            k = i % 8
            accs[k] = vals if accs[k] is None else accs[k] * vals
        acc = ((accs[0] * accs[1]) * (accs[2] * accs[3])) * (
            (accs[4] * accs[5]) * (accs[6] * accs[7])
        )
        out_v[pl.ds(g * _LN, _LN)] = jnp.clip(1.0 - acc, 1e-4, 1.0 - 1e-5)
        return c

    lax.fori_loop(0, _ROWS // _LN, group, 0)

    pltpu.sync_copy(out_v, out_hbm.at[pl.ds(base, _ROWS)])


def kernel(rules, W):
    rules_flat = rules.reshape(-1).astype(jnp.int32)
    wp = jnp.concatenate(
        [W.reshape(-1).astype(jnp.float32),
         jnp.zeros((_TAB - _PAD - 1,), jnp.float32)]
    )
    f = pl.kernel(
        _noisy_or_body,
        mesh=plsc.VectorSubcoreMesh(core_axis_name="c", subcore_axis_name="s"),
        compiler_params=pltpu.CompilerParams(needs_layout_passes=False),
        out_type=jax.ShapeDtypeStruct((_B,), jnp.float32),
        scratch_types=[
            pltpu.VMEM((_ROWS * _L,), jnp.int32),
            pltpu.VMEM((_TAB,), jnp.float32),
            pltpu.VMEM((_ROWS,), jnp.float32),
            pltpu.SemaphoreType.DMA,
        ],
    )
    return f(rules_flat, wp).reshape(_B, 1)
